# baseline (speedup 1.0000x reference)
import functools

import jax
import jax.numpy as jnp
from jax.experimental import pallas as pl
from jax.experimental.pallas import tpu as pltpu

NUM_NODES = 100000
N_SUB = 20000
E = 200000
B = 4096
MEM_DIM = 100
TIME_DIM = 100
EMB_DIM = 100
MSG_DIM = 16
HEADS = 2
HEAD_DIM = EMB_DIM // HEADS


def _linkpred_body(zs_ref, zd_ref, zn_ref, Wls_ref, bls_ref, Wld_ref, bld_ref,
                   Wlf_ref, blf_ref, pos_ref, neg_ref):
    zs = zs_ref[...]
    zd = zd_ref[...]
    zn = zn_ref[...]
    a = jnp.dot(zs, Wls_ref[...], preferred_element_type=jnp.float32) + bls_ref[...]
    hd = jnp.dot(zd, Wld_ref[...], preferred_element_type=jnp.float32) + bld_ref[...]
    hn = jnp.dot(zn, Wld_ref[...], preferred_element_type=jnp.float32) + bld_ref[...]
    h = jnp.maximum(a + hd, 0.0)
    hneg = jnp.maximum(a + hn, 0.0)
    pos_ref[...] = jnp.dot(h, Wlf_ref[...], preferred_element_type=jnp.float32) + blf_ref[...]
    neg_ref[...] = jnp.dot(hneg, Wlf_ref[...], preferred_element_type=jnp.float32) + blf_ref[...]


def _linkpred(z_src, z_dst, z_ndst, Wls, bls, Wld, bld, Wlf, blf):
    out_shape = (jax.ShapeDtypeStruct((B, 1), jnp.float32),
                 jax.ShapeDtypeStruct((B, 1), jnp.float32))
    return pl.pallas_call(
        _linkpred_body,
        out_shape=out_shape,
    )(z_src, z_dst, z_ndst, Wls, bls[None, :], Wld, bld[None, :], Wlf, blf[None, :])


def kernel(memory, last_update, t, msg, w_time, b_time, Wq, bq, Wk, bk, Wv, bv,
           We, be, Ws, bs, Wls, bls, Wld, bld, Wlf, blf,
           n_id, edge_index, src, dst, neg_dst):
    assoc = jnp.zeros((NUM_NODES,), jnp.int32).at[n_id].set(
        jnp.arange(N_SUB, dtype=jnp.int32))
    z0 = memory[n_id]
    lu = last_update[n_id]
    src_e = edge_index[0]
    dst_e = edge_index[1]
    rel_t = lu[src_e] - t
    rel_t_enc = jnp.cos(rel_t[:, None] @ w_time.T + b_time)
    edge_attr = jnp.concatenate([rel_t_enc, msg], axis=-1)
    q = (z0 @ Wq + bq).reshape(N_SUB, HEADS, HEAD_DIM)
    k = (z0 @ Wk + bk).reshape(N_SUB, HEADS, HEAD_DIM)
    v = (z0 @ Wv + bv).reshape(N_SUB, HEADS, HEAD_DIM)
    e = (edge_attr @ We + be).reshape(E, HEADS, HEAD_DIM)
    kj = k[src_e] + e
    qi = q[dst_e]
    alpha = (qi * kj).sum(-1) / jnp.sqrt(float(HEAD_DIM))
    amax = jax.ops.segment_max(alpha, dst_e, num_segments=N_SUB)
    amax = jnp.where(jnp.isfinite(amax), amax, 0.0)
    ex = jnp.exp(alpha - amax[dst_e])
    denom = jax.ops.segment_sum(ex, dst_e, num_segments=N_SUB)
    alpha = ex / (denom[dst_e] + 1e-16)
    vj = v[src_e] + e
    out_e = (vj * alpha[:, :, None]).reshape(E, HEADS * HEAD_DIM)
    out = jax.ops.segment_sum(out_e, dst_e, num_segments=N_SUB)
    z = out + (z0 @ Ws + bs)
    z_src = z[assoc[src]]
    z_dst = z[assoc[dst]]
    z_ndst = z[assoc[neg_dst]]
    pos_out, neg_out = _linkpred(z_src, z_dst, z_ndst, Wls, bls, Wld, bld, Wlf, blf)
    return (z, pos_out, neg_out)


# speedup vs baseline: 2.9003x; 2.9003x over previous
import functools

import jax
import jax.numpy as jnp
from jax import lax
from jax.experimental import pallas as pl
from jax.experimental.pallas import tpu as pltpu
from jax.experimental.pallas import tpu_sc as plsc

NUM_NODES = 100000
N_SUB = 20000
E = 200000
B = 4096
MEM_DIM = 100
TIME_DIM = 100
EMB_DIM = 100
MSG_DIM = 16
HEADS = 2
HEAD_DIM = EMB_DIM // HEADS
NQ = 3 * B

NODE_BLK = 2000
EDGE_BLK = 2000

_DBG_JAX_SCE = False
_DBG_JAX_SC1A = False
_DBG_JAX_SC1B = False
_DBG_Z0_JAX = False
_DBG_JAX_SC2 = False

_ROW_CHUNKS = (N_SUB + 127) // 128
_REL_CHUNKS = (E + 511) // 512
_SCE_CHUNKS = (E + 127) // 128


def _mesh():
    return plsc.VectorSubcoreMesh(core_axis_name="c", subcore_axis_name="s")


_SC_PARAMS = pltpu.CompilerParams(needs_layout_passes=False,
                                  use_tc_tiling_on_sc=False)


def _pad_body(m_ref, o_ref):
    blk = m_ref.shape[0]
    o_ref[...] = jnp.concatenate(
        [m_ref[...], jnp.zeros((blk, 12), jnp.float32)], axis=1)


def _pad112(memory):
    blk = 2000
    return pl.pallas_call(
        _pad_body,
        grid=(NUM_NODES // blk,),
        in_specs=[pl.BlockSpec((blk, MEM_DIM), lambda i: (i, 0))],
        out_specs=pl.BlockSpec((blk, 112), lambda i: (i, 0)),
        out_shape=jax.ShapeDtypeStruct((NUM_NODES, 112), jnp.float32),
    )(memory)


def _sc1a_body(mem_hbm, nid_hbm, qry_hbm,
               z0_hbm, loc_hbm,
               idx_v, row_v, chq_v, outq_v, assoc_v, sem):
    c = lax.axis_index("c")
    s = lax.axis_index("s")
    wid = s * 2 + c

    def chunk(g, _):
        gi = jnp.minimum(wid + 32 * g, _ROW_CHUNKS - 1)
        base = jnp.minimum(gi * 128, N_SUB - 128)
        pltpu.sync_copy(nid_hbm.at[pl.ds(base, 128)], idx_v)
        pltpu.async_copy(mem_hbm.at[idx_v], row_v, sem).wait()
        pltpu.sync_copy(row_v, z0_hbm.at[pl.ds(base, 128), :])
        return _
    lax.fori_loop(0, (_ROW_CHUNKS + 31) // 32, chunk, None, unroll=False)

    @pl.when(jnp.logical_and(c == 0, s == 0))
    def _assoc():
        zero16 = jnp.zeros((16,), jnp.int32)

        def zloop(i, _):
            assoc_v[pl.ds(i * 16, 16)] = zero16
            return _
        lax.fori_loop(0, NUM_NODES // 16, zloop, None, unroll=False)

        iota16 = lax.iota(jnp.int32, 16)

        def achunk(g, _):
            pltpu.sync_copy(nid_hbm.at[pl.ds(g * 512, 512)], chq_v)

            def grp(i, _):
                gv = chq_v[pl.ds(i * 16, 16)]
                iv = g * 512 + i * 16 + iota16
                for l in range(16):
                    plsc.store_scatter(assoc_v, [gv], iv, mask=(iota16 == l))
                return _
            lax.fori_loop(0, 32, grp, None, unroll=False)
            return _
        lax.fori_loop(0, N_SUB // 512, achunk, None, unroll=False)

        _TB = (N_SUB // 512) * 512
        pltpu.sync_copy(nid_hbm.at[pl.ds(_TB, N_SUB - _TB)],
                        chq_v.at[pl.ds(0, N_SUB - _TB)])

        def tgrp(i, _):
            gv = chq_v[pl.ds(i * 16, 16)]
            iv = _TB + i * 16 + iota16
            for l in range(16):
                plsc.store_scatter(assoc_v, [gv], iv, mask=(iota16 == l))
            return _
        lax.fori_loop(0, (N_SUB - _TB) // 16, tgrp, None, unroll=False)

        def qchunk(g, _):
            pltpu.sync_copy(qry_hbm.at[pl.ds(g * 512, 512)], chq_v)

            def vec(i, _):
                qv = chq_v[pl.ds(i * 16, 16)]
                outq_v[pl.ds(i * 16, 16)] = plsc.load_gather(assoc_v, [qv])
                return _
            lax.fori_loop(0, 32, vec, None, unroll=False)
            pltpu.sync_copy(outq_v, loc_hbm.at[pl.ds(g * 512, 512)])
            return _
        lax.fori_loop(0, NQ // 512, qchunk, None, unroll=False)


def _sc1a(memory, n_id, queries):
    f = pl.kernel(
        _sc1a_body,
        out_type=(
            jax.ShapeDtypeStruct((N_SUB, 112), jnp.float32),
            jax.ShapeDtypeStruct((NQ,), jnp.int32),
        ),
        mesh=_mesh(),
        compiler_params=_SC_PARAMS,
        scratch_types=[
            pltpu.VMEM((128,), jnp.int32),
            pltpu.VMEM((128, 112), jnp.float32),
            pltpu.VMEM((512,), jnp.int32),
            pltpu.VMEM((512,), jnp.int32),
            pltpu.VMEM((NUM_NODES,), jnp.int32),
            pltpu.SemaphoreType.DMA,
        ],
    )
    return f(memory, n_id, queries)


def _sc1b_body(lu_hbm, nid_hbm, srce_hbm, t_hbm,
               rel_hbm,
               nid_v, lu_v, chs_v, cht_v, out_v):
    c = lax.axis_index("c")
    s = lax.axis_index("s")
    wid = s * 2 + c
    pltpu.sync_copy(nid_hbm, nid_v)
    pltpu.sync_copy(lu_hbm, lu_v)

    def chunk(g, _):
        gi = jnp.minimum(wid + 32 * g, _REL_CHUNKS - 1)
        base = jnp.minimum(gi * 512, E - 512)
        pltpu.sync_copy(srce_hbm.at[pl.ds(base, 512)], chs_v)
        pltpu.sync_copy(t_hbm.at[pl.ds(base, 512)], cht_v)

        def vec(i, _):
            srcv = chs_v[pl.ds(i * 16, 16)]
            gid = plsc.load_gather(nid_v, [srcv])
            luv = plsc.load_gather(lu_v, [gid])
            out_v[pl.ds(i * 16, 16)] = luv - cht_v[pl.ds(i * 16, 16)]
            return _
        lax.fori_loop(0, 32, vec, None, unroll=False)
        pltpu.sync_copy(out_v, rel_hbm.at[pl.ds(base, 512)])
        return _
    lax.fori_loop(0, (_REL_CHUNKS + 31) // 32, chunk, None, unroll=False)


def _sc1b(last_update, n_id, src_e, t):
    f = pl.kernel(
        _sc1b_body,
        out_type=jax.ShapeDtypeStruct((E,), jnp.float32),
        mesh=_mesh(),
        compiler_params=_SC_PARAMS,
        scratch_types=[
            pltpu.VMEM((N_SUB,), jnp.int32),
            pltpu.VMEM((NUM_NODES,), jnp.float32),
            pltpu.VMEM((512,), jnp.int32),
            pltpu.VMEM((512,), jnp.float32),
            pltpu.VMEM((512,), jnp.float32),
        ],
    )
    return f(last_update, n_id, src_e, t)


def _nodeproj_body(z0_ref, Wq_ref, bq_ref, Wk_ref, bk_ref, Wv_ref, bv_ref,
                   Ws_ref, bs_ref,
                   kv0_ref, kv1_ref, qs0_ref, qs1_ref, zs_ref):
    z0 = z0_ref[...][:, :MEM_DIM]
    n = z0.shape[0]
    zpad = jnp.zeros((n, 14), jnp.float32)
    q = jnp.dot(z0, Wq_ref[...], preferred_element_type=jnp.float32) + bq_ref[...]
    k = jnp.dot(z0, Wk_ref[...], preferred_element_type=jnp.float32) + bk_ref[...]
    v = jnp.dot(z0, Wv_ref[...], preferred_element_type=jnp.float32) + bv_ref[...]
    isq = 1.0 / (float(HEAD_DIM) ** 0.5)
    kv0_ref[...] = jnp.concatenate(
        [k[:, :HEAD_DIM], zpad, v[:, :HEAD_DIM], zpad], axis=1)
    kv1_ref[...] = jnp.concatenate(
        [k[:, HEAD_DIM:], zpad, v[:, HEAD_DIM:], zpad], axis=1)
    qs0_ref[...] = jnp.concatenate([q[:, :HEAD_DIM] * isq, zpad], axis=1)
    qs1_ref[...] = jnp.concatenate([q[:, HEAD_DIM:] * isq, zpad], axis=1)
    zs_ref[...] = jnp.dot(z0, Ws_ref[...], preferred_element_type=jnp.float32) + bs_ref[...]


def _nodeproj(z0, Wq, bq, Wk, bk, Wv, bv, Ws, bs):
    nblk = N_SUB // NODE_BLK
    w_spec = pl.BlockSpec((MEM_DIM, EMB_DIM), lambda i: (0, 0))
    b_spec = pl.BlockSpec((1, EMB_DIM), lambda i: (0, 0))
    row_spec = pl.BlockSpec((NODE_BLK, EMB_DIM), lambda i: (i, 0))
    z0_spec = pl.BlockSpec((NODE_BLK, 112), lambda i: (i, 0))
    kv_spec = pl.BlockSpec((NODE_BLK, 128), lambda i: (i, 0))
    q_spec = pl.BlockSpec((NODE_BLK, 64), lambda i: (i, 0))
    out_shape = (
        jax.ShapeDtypeStruct((N_SUB, 128), jnp.float32),
        jax.ShapeDtypeStruct((N_SUB, 128), jnp.float32),
        jax.ShapeDtypeStruct((N_SUB, 64), jnp.float32),
        jax.ShapeDtypeStruct((N_SUB, 64), jnp.float32),
        jax.ShapeDtypeStruct((N_SUB, EMB_DIM), jnp.float32),
    )
    return pl.pallas_call(
        _nodeproj_body,
        grid=(nblk,),
        in_specs=[z0_spec, w_spec, b_spec, w_spec, b_spec, w_spec, b_spec,
                  w_spec, b_spec],
        out_specs=(kv_spec, kv_spec, q_spec, q_spec, row_spec),
        out_shape=out_shape,
    )(z0, Wq, bq[None, :], Wk, bk[None, :], Wv, bv[None, :], Ws, bs[None, :])


def _edgefeat_body(rel_ref, msg_ref, wrow_ref, brow_ref, Wet_ref, Wem_ref,
                   be_ref, ef0_ref, ef1_ref):
    rel = rel_ref[...]
    n = rel.shape[0]
    enc = jnp.cos(rel * wrow_ref[...] + brow_ref[...])
    e = jnp.dot(enc, Wet_ref[...], preferred_element_type=jnp.float32)
    e = e + jnp.dot(msg_ref[...], Wem_ref[...], preferred_element_type=jnp.float32)
    e = e + be_ref[...]
    ones = jnp.ones((n, 1), jnp.float32)
    zpad = jnp.zeros((n, 13), jnp.float32)
    ef0_ref[...] = jnp.concatenate([e[:, :HEAD_DIM], ones, zpad], axis=1)
    ef1_ref[...] = jnp.concatenate([e[:, HEAD_DIM:], ones, zpad], axis=1)


def _edgefeat(rel_t, msg, w_time, b_time, We, be):
    eblk = E // EDGE_BLK
    ef_spec = pl.BlockSpec((EDGE_BLK, 64), lambda i: (i, 0))
    out_shape = (jax.ShapeDtypeStruct((E, 64), jnp.float32),
                 jax.ShapeDtypeStruct((E, 64), jnp.float32))
    return pl.pallas_call(
        _edgefeat_body,
        grid=(eblk,),
        in_specs=[
            pl.BlockSpec((EDGE_BLK, 1), lambda i: (i, 0)),
            pl.BlockSpec((EDGE_BLK, MSG_DIM), lambda i: (i, 0)),
            pl.BlockSpec((1, TIME_DIM), lambda i: (0, 0)),
            pl.BlockSpec((1, TIME_DIM), lambda i: (0, 0)),
            pl.BlockSpec((TIME_DIM, EMB_DIM), lambda i: (0, 0)),
            pl.BlockSpec((MSG_DIM, EMB_DIM), lambda i: (0, 0)),
            pl.BlockSpec((1, EMB_DIM), lambda i: (0, 0)),
        ],
        out_specs=(ef_spec, ef_spec),
        out_shape=out_shape,
    )(rel_t[:, None], msg, w_time.T, b_time[None, :], We[:TIME_DIM],
      We[TIME_DIM:], be[None, :])


def _sce_head(kv_hbm, qs_hbm, ef_hbm, srce_hbm, dste_hbm, zero_hbm, acc_hbm,
              srcb, dstb, srcb_t, dstb_t, kvb, qb, eb, ob, sem, acc_sh, s):
    r0 = s * (N_SUB // 16)
    pltpu.sync_copy(zero_hbm.at[pl.ds(r0, N_SUB // 16), :],
                    acc_sh.at[pl.ds(r0, N_SUB // 16), :])
    plsc.subcore_barrier()

    def process(base, n, sb, db):
        pltpu.sync_copy(srce_hbm.at[pl.ds(base, n)], sb)
        pltpu.sync_copy(dste_hbm.at[pl.ds(base, n)], db)
        pltpu.async_copy(kv_hbm.at[sb], kvb.at[pl.ds(0, n), :], sem).wait()
        pltpu.async_copy(qs_hbm.at[db], qb.at[pl.ds(0, n), :], sem).wait()
        pltpu.sync_copy(ef_hbm.at[pl.ds(base, n), :], eb.at[pl.ds(0, n), :])

        def edge(i, _):
            la = jnp.zeros((16,), jnp.float32)
            for j in range(4):
                kj = kvb[i, pl.ds(16 * j, 16)]
                ej = eb[i, pl.ds(16 * j, 16)]
                qj = qb[i, pl.ds(16 * j, 16)]
                la = la + qj * (kj + ej)
            logit = jnp.sum(la)
            pvec = jnp.exp(jnp.full((16,), logit, jnp.float32))
            for j in range(4):
                vj = kvb[i, pl.ds(64 + 16 * j, 16)]
                ej = eb[i, pl.ds(16 * j, 16)]
                ob[i, pl.ds(16 * j, 16)] = (vj + ej) * pvec
            return _
        lax.fori_loop(0, n, edge, None, unroll=False)
        pltpu.sync_copy(ob.at[pl.ds(0, n), :], acc_sh.at[db], add=True)

    _FULL = E // 128
    def chunk(g, _):
        gi = s + 16 * g

        @pl.when(gi < _FULL)
        def _():
            process(gi * 128, 128, srcb, dstb)
        return _
    lax.fori_loop(0, (_FULL + 15) // 16, chunk, None, unroll=False)

    @pl.when(s == 0)
    def _tail():
        process(_FULL * 128, E - _FULL * 128, srcb_t, dstb_t)

    plsc.subcore_barrier()
    pltpu.sync_copy(acc_sh.at[pl.ds(r0, N_SUB // 16), :],
                    acc_hbm.at[pl.ds(r0, N_SUB // 16), :])


def _sce_body(kv0, kv1, qs0, qs1, ef0, ef1, srce, dste, zeros,
              acc0, acc1,
              srcb, dstb, srcb_t, dstb_t, kvb, qb, eb, ob, sem, acc_sh):
    c = lax.axis_index("c")
    s = lax.axis_index("s")

    @pl.when(c == 0)
    def _h0():
        _sce_head(kv0, qs0, ef0, srce, dste, zeros, acc0,
                  srcb, dstb, srcb_t, dstb_t, kvb, qb, eb, ob, sem, acc_sh, s)

    @pl.when(c == 1)
    def _h1():
        _sce_head(kv1, qs1, ef1, srce, dste, zeros, acc1,
                  srcb, dstb, srcb_t, dstb_t, kvb, qb, eb, ob, sem, acc_sh, s)


def _sce(kv0, kv1, qs0, qs1, ef0, ef1, src_e, dst_e):
    zeros = jnp.zeros((N_SUB, 64), jnp.float32)
    f = pl.kernel(
        _sce_body,
        out_type=(jax.ShapeDtypeStruct((N_SUB, 64), jnp.float32),
                  jax.ShapeDtypeStruct((N_SUB, 64), jnp.float32)),
        mesh=_mesh(),
        compiler_params=_SC_PARAMS,
        scratch_types=[
            pltpu.VMEM((128,), jnp.int32),
            pltpu.VMEM((128,), jnp.int32),
            pltpu.VMEM((64,), jnp.int32),
            pltpu.VMEM((64,), jnp.int32),
            pltpu.VMEM((128, 128), jnp.float32),
            pltpu.VMEM((128, 64), jnp.float32),
            pltpu.VMEM((128, 64), jnp.float32),
            pltpu.VMEM((128, 64), jnp.float32),
            pltpu.SemaphoreType.DMA,
            pltpu.VMEM_SHARED((N_SUB, 64), jnp.float32),
        ],
    )
    return f(kv0, kv1, qs0, qs1, ef0, ef1, src_e, dst_e, zeros)


def _fin_body(a0_ref, a1_ref, zs_ref, z_ref, zp_ref):
    a0 = a0_ref[...]
    a1 = a1_ref[...]
    s0 = a0[:, HEAD_DIM:HEAD_DIM + 1]
    s1 = a1[:, HEAD_DIM:HEAD_DIM + 1]
    h0 = a0[:, :HEAD_DIM] / jnp.where(s0 > 0, s0, 1.0)
    h1 = a1[:, :HEAD_DIM] / jnp.where(s1 > 0, s1, 1.0)
    z = jnp.concatenate([h0, h1], axis=1) + zs_ref[...]
    z_ref[...] = z
    zp_ref[...] = jnp.concatenate(
        [z, jnp.zeros((z.shape[0], 12), jnp.float32)], axis=1)


def _finalize(acc0, acc1, zs):
    nblk = N_SUB // NODE_BLK
    return pl.pallas_call(
        _fin_body,
        grid=(nblk,),
        in_specs=[
            pl.BlockSpec((NODE_BLK, 64), lambda i: (i, 0)),
            pl.BlockSpec((NODE_BLK, 64), lambda i: (i, 0)),
            pl.BlockSpec((NODE_BLK, EMB_DIM), lambda i: (i, 0)),
        ],
        out_specs=(pl.BlockSpec((NODE_BLK, EMB_DIM), lambda i: (i, 0)),
                   pl.BlockSpec((NODE_BLK, 112), lambda i: (i, 0))),
        out_shape=(jax.ShapeDtypeStruct((N_SUB, EMB_DIM), jnp.float32),
                   jax.ShapeDtypeStruct((N_SUB, 112), jnp.float32)),
    )(acc0, acc1, zs)


def _sc2_body(z_hbm, loc_hbm, zsel_hbm, idx_v, row_v, sem):
    c = lax.axis_index("c")
    s = lax.axis_index("s")
    wid = s * 2 + c

    def chunk(g, _):
        base = (wid * 3 + g) * 128
        pltpu.sync_copy(loc_hbm.at[pl.ds(base, 128)], idx_v)
        pltpu.async_copy(z_hbm.at[idx_v], row_v, sem).wait()
        pltpu.sync_copy(row_v, zsel_hbm.at[pl.ds(base, 128), :])
        return _
    lax.fori_loop(0, 3, chunk, None, unroll=False)


def _sc2(z, loc):
    f = pl.kernel(
        _sc2_body,
        out_type=jax.ShapeDtypeStruct((NQ, 112), jnp.float32),
        mesh=_mesh(),
        compiler_params=_SC_PARAMS,
        scratch_types=[
            pltpu.VMEM((128,), jnp.int32),
            pltpu.VMEM((128, 112), jnp.float32),
            pltpu.SemaphoreType.DMA,
        ],
    )
    return f(z, loc)


def _linkpred_body(zs_ref, zd_ref, zn_ref, Wls_ref, bls_ref, Wld_ref, bld_ref,
                   Wlf_ref, blf_ref, pos_ref, neg_ref):
    zsv = zs_ref[...][:, :EMB_DIM]
    zdv = zd_ref[...][:, :EMB_DIM]
    znv = zn_ref[...][:, :EMB_DIM]
    a = jnp.dot(zsv, Wls_ref[...], preferred_element_type=jnp.float32) + bls_ref[...]
    hd = jnp.dot(zdv, Wld_ref[...], preferred_element_type=jnp.float32) + bld_ref[...]
    hn = jnp.dot(znv, Wld_ref[...], preferred_element_type=jnp.float32) + bld_ref[...]
    h = jnp.maximum(a + hd, 0.0)
    hneg = jnp.maximum(a + hn, 0.0)
    pos_ref[...] = jnp.dot(h, Wlf_ref[...], preferred_element_type=jnp.float32) + blf_ref[...]
    neg_ref[...] = jnp.dot(hneg, Wlf_ref[...], preferred_element_type=jnp.float32) + blf_ref[...]


def _linkpred(zsel, Wls, bls, Wld, bld, Wlf, blf):
    out_shape = (jax.ShapeDtypeStruct((B, 1), jnp.float32),
                 jax.ShapeDtypeStruct((B, 1), jnp.float32))
    return pl.pallas_call(
        _linkpred_body,
        out_shape=out_shape,
    )(zsel[:B], zsel[B:2 * B], zsel[2 * B:], Wls, bls[None, :], Wld,
      bld[None, :], Wlf, blf[None, :])


def kernel(memory, last_update, t, msg, w_time, b_time, Wq, bq, Wk, bk, Wv, bv,
           We, be, Ws, bs, Wls, bls, Wld, bld, Wlf, blf,
           n_id, edge_index, src, dst, neg_dst):
    src_e = edge_index[0]
    dst_e = edge_index[1]
    queries = jnp.concatenate([src, dst, neg_dst])

    memp = _pad112(memory)
    if _DBG_JAX_SC1A:
        z0 = jnp.pad(memory[n_id], ((0, 0), (0, 12)))
        assoc = jnp.zeros((NUM_NODES,), jnp.int32).at[n_id].set(
            jnp.arange(N_SUB, dtype=jnp.int32))
        loc = assoc[queries]
    else:
        z0, loc = _sc1a(memp, n_id, queries)
        if _DBG_Z0_JAX:
            z0 = jnp.pad(memory[n_id], ((0, 0), (0, 12)))
    if _DBG_JAX_SC1B:
        rel_t = last_update[n_id][src_e] - t
    else:
        rel_t = _sc1b(last_update, n_id, src_e, t)
    kv0, kv1, qs0, qs1, zs = _nodeproj(z0, Wq, bq, Wk, bk, Wv, bv, Ws, bs)
    ef0, ef1 = _edgefeat(rel_t, msg, w_time, b_time, We, be)
    if _DBG_JAX_SCE:
        def sce_ref(kv, qs, ef):
            kk = kv[:, :64][src_e]
            vv = kv[:, 64:][src_e]
            qq = qs[dst_e]
            p = jnp.exp((qq * (kk + ef)).sum(-1))
            return jax.ops.segment_sum((vv + ef) * p[:, None], dst_e,
                                       num_segments=N_SUB)
        acc0 = sce_ref(kv0, qs0, ef0)
        acc1 = sce_ref(kv1, qs1, ef1)
    else:
        acc0, acc1 = _sce(kv0, kv1, qs0, qs1, ef0, ef1, src_e, dst_e)
    z, zp = _finalize(acc0, acc1, zs)
    if _DBG_JAX_SC2:
        zsel = zp[loc]
    else:
        zsel = _sc2(zp, loc)
    pos_out, neg_out = _linkpred(zsel, Wls, bls, Wld, bld, Wlf, blf)
    return (z, pos_out, neg_out)


# speedup vs baseline: 3.1647x; 1.0911x over previous
import functools

import jax
import jax.numpy as jnp
from jax import lax
from jax.experimental import pallas as pl
from jax.experimental.pallas import tpu as pltpu
from jax.experimental.pallas import tpu_sc as plsc

NUM_NODES = 100000
N_SUB = 20000
E = 200000
B = 4096
MEM_DIM = 100
TIME_DIM = 100
EMB_DIM = 100
MSG_DIM = 16
HEADS = 2
HEAD_DIM = EMB_DIM // HEADS
NQ = 3 * B

NODE_BLK = 2000
EDGE_BLK = 2000

_DBG_JAX_SCE = False
_DBG_JAX_SC1A = False
_DBG_JAX_SC1B = False
_DBG_Z0_JAX = False
_DBG_JAX_SC2 = False

_ROW_CHUNKS = (N_SUB + 127) // 128
_REL_CHUNKS = (E + 511) // 512
_SCE_CHUNKS = (E + 127) // 128


def _mesh():
    return plsc.VectorSubcoreMesh(core_axis_name="c", subcore_axis_name="s")


_SC_PARAMS = pltpu.CompilerParams(needs_layout_passes=False,
                                  use_tc_tiling_on_sc=False)


def _pad_body(m_ref, o_ref):
    blk = m_ref.shape[0]
    o_ref[...] = jnp.concatenate(
        [m_ref[...], jnp.zeros((blk, 12), jnp.float32)], axis=1)


def _pad112(memory):
    blk = 2000
    return pl.pallas_call(
        _pad_body,
        grid=(NUM_NODES // blk,),
        in_specs=[pl.BlockSpec((blk, MEM_DIM), lambda i: (i, 0))],
        out_specs=pl.BlockSpec((blk, 112), lambda i: (i, 0)),
        out_shape=jax.ShapeDtypeStruct((NUM_NODES, 112), jnp.float32),
    )(memory)


def _sc1a_body(mem_hbm, nid_hbm, qry_hbm,
               z0_hbm, loc_hbm,
               idx_v, row_v, chq_v, outq_v, assoc_v, sem):
    c = lax.axis_index("c")
    s = lax.axis_index("s")
    wid = s * 2 + c

    def chunk(g, _):
        gi = jnp.minimum(wid + 32 * g, _ROW_CHUNKS - 1)
        base = jnp.minimum(gi * 128, N_SUB - 128)
        pltpu.sync_copy(nid_hbm.at[pl.ds(base, 128)], idx_v)
        pltpu.async_copy(mem_hbm.at[idx_v], row_v, sem).wait()
        pltpu.sync_copy(row_v, z0_hbm.at[pl.ds(base, 128), :])
        return _
    lax.fori_loop(0, (_ROW_CHUNKS + 31) // 32, chunk, None, unroll=False)

    @pl.when(jnp.logical_and(c == 0, s == 0))
    def _assoc():
        zero16 = jnp.zeros((16,), jnp.int32)

        def zloop(i, _):
            assoc_v[pl.ds(i * 16, 16)] = zero16
            return _
        lax.fori_loop(0, NUM_NODES // 16, zloop, None, unroll=False)

        iota16 = lax.iota(jnp.int32, 16)

        def achunk(g, _):
            pltpu.sync_copy(nid_hbm.at[pl.ds(g * 512, 512)], chq_v)

            def grp(i, _):
                gv = chq_v[pl.ds(i * 16, 16)]
                iv = g * 512 + i * 16 + iota16
                for l in range(16):
                    plsc.store_scatter(assoc_v, [gv], iv, mask=(iota16 == l))
                return _
            lax.fori_loop(0, 32, grp, None, unroll=False)
            return _
        lax.fori_loop(0, N_SUB // 512, achunk, None, unroll=False)

        _TB = (N_SUB // 512) * 512
        pltpu.sync_copy(nid_hbm.at[pl.ds(_TB, N_SUB - _TB)],
                        chq_v.at[pl.ds(0, N_SUB - _TB)])

        def tgrp(i, _):
            gv = chq_v[pl.ds(i * 16, 16)]
            iv = _TB + i * 16 + iota16
            for l in range(16):
                plsc.store_scatter(assoc_v, [gv], iv, mask=(iota16 == l))
            return _
        lax.fori_loop(0, (N_SUB - _TB) // 16, tgrp, None, unroll=False)

        def qchunk(g, _):
            pltpu.sync_copy(qry_hbm.at[pl.ds(g * 512, 512)], chq_v)

            def vec(i, _):
                qv = chq_v[pl.ds(i * 16, 16)]
                outq_v[pl.ds(i * 16, 16)] = plsc.load_gather(assoc_v, [qv])
                return _
            lax.fori_loop(0, 32, vec, None, unroll=False)
            pltpu.sync_copy(outq_v, loc_hbm.at[pl.ds(g * 512, 512)])
            return _
        lax.fori_loop(0, NQ // 512, qchunk, None, unroll=False)


def _sc1a(memory, n_id, queries):
    f = pl.kernel(
        _sc1a_body,
        out_type=(
            jax.ShapeDtypeStruct((N_SUB, 112), jnp.float32),
            jax.ShapeDtypeStruct((NQ,), jnp.int32),
        ),
        mesh=_mesh(),
        compiler_params=_SC_PARAMS,
        scratch_types=[
            pltpu.VMEM((128,), jnp.int32),
            pltpu.VMEM((128, 112), jnp.float32),
            pltpu.VMEM((512,), jnp.int32),
            pltpu.VMEM((512,), jnp.int32),
            pltpu.VMEM((NUM_NODES,), jnp.int32),
            pltpu.SemaphoreType.DMA,
        ],
    )
    return f(memory, n_id, queries)


def _sc1b_body(lu_hbm, nid_hbm, srce_hbm, t_hbm,
               rel_hbm,
               nid_v, lu_v, chs_v, cht_v, out_v):
    c = lax.axis_index("c")
    s = lax.axis_index("s")
    wid = s * 2 + c
    pltpu.sync_copy(nid_hbm, nid_v)
    pltpu.sync_copy(lu_hbm, lu_v)

    def chunk(g, _):
        gi = jnp.minimum(wid + 32 * g, _REL_CHUNKS - 1)
        base = jnp.minimum(gi * 512, E - 512)
        pltpu.sync_copy(srce_hbm.at[pl.ds(base, 512)], chs_v)
        pltpu.sync_copy(t_hbm.at[pl.ds(base, 512)], cht_v)

        def vec(i, _):
            srcv = chs_v[pl.ds(i * 16, 16)]
            gid = plsc.load_gather(nid_v, [srcv])
            luv = plsc.load_gather(lu_v, [gid])
            out_v[pl.ds(i * 16, 16)] = luv - cht_v[pl.ds(i * 16, 16)]
            return _
        lax.fori_loop(0, 32, vec, None, unroll=False)
        pltpu.sync_copy(out_v, rel_hbm.at[pl.ds(base, 512)])
        return _
    lax.fori_loop(0, (_REL_CHUNKS + 31) // 32, chunk, None, unroll=False)


def _sc1b(last_update, n_id, src_e, t):
    f = pl.kernel(
        _sc1b_body,
        out_type=jax.ShapeDtypeStruct((E,), jnp.float32),
        mesh=_mesh(),
        compiler_params=_SC_PARAMS,
        scratch_types=[
            pltpu.VMEM((N_SUB,), jnp.int32),
            pltpu.VMEM((NUM_NODES,), jnp.float32),
            pltpu.VMEM((512,), jnp.int32),
            pltpu.VMEM((512,), jnp.float32),
            pltpu.VMEM((512,), jnp.float32),
        ],
    )
    return f(last_update, n_id, src_e, t)


def _nodeproj_body(z0_ref, Wq_ref, bq_ref, Wk_ref, bk_ref, Wv_ref, bv_ref,
                   Ws_ref, bs_ref,
                   kv0_ref, kv1_ref, qs0_ref, qs1_ref, zs_ref):
    z0 = z0_ref[...][:, :MEM_DIM]
    n = z0.shape[0]
    zpad = jnp.zeros((n, 14), jnp.float32)
    q = jnp.dot(z0, Wq_ref[...], preferred_element_type=jnp.float32) + bq_ref[...]
    k = jnp.dot(z0, Wk_ref[...], preferred_element_type=jnp.float32) + bk_ref[...]
    v = jnp.dot(z0, Wv_ref[...], preferred_element_type=jnp.float32) + bv_ref[...]
    isq = 1.0 / (float(HEAD_DIM) ** 0.5)
    kv0_ref[...] = jnp.concatenate(
        [k[:, :HEAD_DIM], zpad, v[:, :HEAD_DIM], zpad], axis=1)
    kv1_ref[...] = jnp.concatenate(
        [k[:, HEAD_DIM:], zpad, v[:, HEAD_DIM:], zpad], axis=1)
    qs0_ref[...] = jnp.concatenate([q[:, :HEAD_DIM] * isq, zpad], axis=1)
    qs1_ref[...] = jnp.concatenate([q[:, HEAD_DIM:] * isq, zpad], axis=1)
    zs_ref[...] = jnp.dot(z0, Ws_ref[...], preferred_element_type=jnp.float32) + bs_ref[...]


def _nodeproj(z0, Wq, bq, Wk, bk, Wv, bv, Ws, bs):
    nblk = N_SUB // NODE_BLK
    w_spec = pl.BlockSpec((MEM_DIM, EMB_DIM), lambda i: (0, 0))
    b_spec = pl.BlockSpec((1, EMB_DIM), lambda i: (0, 0))
    row_spec = pl.BlockSpec((NODE_BLK, EMB_DIM), lambda i: (i, 0))
    z0_spec = pl.BlockSpec((NODE_BLK, 112), lambda i: (i, 0))
    kv_spec = pl.BlockSpec((NODE_BLK, 128), lambda i: (i, 0))
    q_spec = pl.BlockSpec((NODE_BLK, 64), lambda i: (i, 0))
    out_shape = (
        jax.ShapeDtypeStruct((N_SUB, 128), jnp.float32),
        jax.ShapeDtypeStruct((N_SUB, 128), jnp.float32),
        jax.ShapeDtypeStruct((N_SUB, 64), jnp.float32),
        jax.ShapeDtypeStruct((N_SUB, 64), jnp.float32),
        jax.ShapeDtypeStruct((N_SUB, EMB_DIM), jnp.float32),
    )
    return pl.pallas_call(
        _nodeproj_body,
        grid=(nblk,),
        in_specs=[z0_spec, w_spec, b_spec, w_spec, b_spec, w_spec, b_spec,
                  w_spec, b_spec],
        out_specs=(kv_spec, kv_spec, q_spec, q_spec, row_spec),
        out_shape=out_shape,
    )(z0, Wq, bq[None, :], Wk, bk[None, :], Wv, bv[None, :], Ws, bs[None, :])


def _edgefeat_body(rel_ref, msg_ref, wrow_ref, brow_ref, Wet_ref, Wem_ref,
                   be_ref, ef0_ref, ef1_ref):
    rel = rel_ref[...]
    n = rel.shape[0]
    enc = jnp.cos(rel * wrow_ref[...] + brow_ref[...])
    e = jnp.dot(enc, Wet_ref[...], preferred_element_type=jnp.float32)
    e = e + jnp.dot(msg_ref[...], Wem_ref[...], preferred_element_type=jnp.float32)
    e = e + be_ref[...]
    ones = jnp.ones((n, 1), jnp.float32)
    zpad = jnp.zeros((n, 13), jnp.float32)
    ef0_ref[...] = jnp.concatenate([e[:, :HEAD_DIM], ones, zpad], axis=1)
    ef1_ref[...] = jnp.concatenate([e[:, HEAD_DIM:], ones, zpad], axis=1)


def _edgefeat(rel_t, msg, w_time, b_time, We, be):
    eblk = E // EDGE_BLK
    ef_spec = pl.BlockSpec((EDGE_BLK, 64), lambda i: (i, 0))
    out_shape = (jax.ShapeDtypeStruct((E, 64), jnp.float32),
                 jax.ShapeDtypeStruct((E, 64), jnp.float32))
    return pl.pallas_call(
        _edgefeat_body,
        grid=(eblk,),
        in_specs=[
            pl.BlockSpec((EDGE_BLK, 1), lambda i: (i, 0)),
            pl.BlockSpec((EDGE_BLK, MSG_DIM), lambda i: (i, 0)),
            pl.BlockSpec((1, TIME_DIM), lambda i: (0, 0)),
            pl.BlockSpec((1, TIME_DIM), lambda i: (0, 0)),
            pl.BlockSpec((TIME_DIM, EMB_DIM), lambda i: (0, 0)),
            pl.BlockSpec((MSG_DIM, EMB_DIM), lambda i: (0, 0)),
            pl.BlockSpec((1, EMB_DIM), lambda i: (0, 0)),
        ],
        out_specs=(ef_spec, ef_spec),
        out_shape=out_shape,
    )(rel_t[:, None], msg, w_time.T, b_time[None, :], We[:TIME_DIM],
      We[TIME_DIM:], be[None, :])


_FULL = E // 128


def _sce_run(kv_hbm, qs_hbm, ef_hbm, srce_hbm, dste_hbm, zero_hbm, acc_hbm,
             srcb, dstb, srcb_t, dstb_t, kvb, qb, eb, ob,
             sems, semd, semk, semq, seme, acc_sh, s):
    r0 = s * (N_SUB // 16)
    pltpu.sync_copy(zero_hbm.at[pl.ds(r0, N_SUB // 16), :],
                    acc_sh.at[pl.ds(r0, N_SUB // 16), :])
    plsc.subcore_barrier()

    def process(base, n, sb, db):
        cs = pltpu.async_copy(srce_hbm.at[pl.ds(base, n)], sb, sems)
        cd = pltpu.async_copy(dste_hbm.at[pl.ds(base, n)], db, semd)
        ce = pltpu.async_copy(ef_hbm.at[pl.ds(base, n), :],
                              eb.at[pl.ds(0, n), :], seme)
        cs.wait()
        cd.wait()
        ck = pltpu.async_copy(kv_hbm.at[sb], kvb.at[pl.ds(0, n), :], semk)
        cq = pltpu.async_copy(qs_hbm.at[db], qb.at[pl.ds(0, n), :], semq)
        ck.wait()
        cq.wait()
        ce.wait()

        def edge(i, _):
            la = jnp.zeros((16,), jnp.float32)
            for j in range(4):
                kj = kvb[i, pl.ds(16 * j, 16)]
                ej = eb[i, pl.ds(16 * j, 16)]
                qj = qb[i, pl.ds(16 * j, 16)]
                la = la + qj * (kj + ej)
            logit = jnp.sum(la)
            pvec = jnp.exp(jnp.full((16,), logit, jnp.float32))
            for j in range(4):
                vj = kvb[i, pl.ds(64 + 16 * j, 16)]
                ej = eb[i, pl.ds(16 * j, 16)]
                ob[i, pl.ds(16 * j, 16)] = (vj + ej) * pvec
            return _
        lax.fori_loop(0, n, edge, None, unroll=False)
        pltpu.sync_copy(ob.at[pl.ds(0, n), :], acc_sh.at[db], add=True)

    def chunk(g, _):
        gi = s + 16 * g

        @pl.when(gi < _FULL)
        def _():
            process(gi * 128, 128, srcb, dstb)
        return _
    lax.fori_loop(0, (_FULL + 15) // 16, chunk, None, unroll=False)

    @pl.when(s == 0)
    def _tail():
        process(_FULL * 128, E - _FULL * 128, srcb_t, dstb_t)

    plsc.subcore_barrier()
    pltpu.sync_copy(acc_sh.at[pl.ds(r0, N_SUB // 16), :],
                    acc_hbm.at[pl.ds(r0, N_SUB // 16), :])


def _sce_body(kv0, kv1, qs0, qs1, ef0, ef1, srce, dste, zeros,
              acc0, acc1,
              srcb, dstb, srcb_t, dstb_t, kvb, qb, eb, ob,
              sems, semd, semk, semq, seme, acc_sh):
    c = lax.axis_index("c")
    s = lax.axis_index("s")

    @pl.when(c == 0)
    def _h0():
        _sce_run(kv0, qs0, ef0, srce, dste, zeros, acc0,
                 srcb, dstb, srcb_t, dstb_t, kvb, qb, eb, ob,
                 sems, semd, semk, semq, seme, acc_sh, s)

    @pl.when(c == 1)
    def _h1():
        _sce_run(kv1, qs1, ef1, srce, dste, zeros, acc1,
                 srcb, dstb, srcb_t, dstb_t, kvb, qb, eb, ob,
                 sems, semd, semk, semq, seme, acc_sh, s)


def _sce(kv0, kv1, qs0, qs1, ef0, ef1, src_e, dst_e):
    zeros = jnp.zeros((N_SUB, 64), jnp.float32)
    f = pl.kernel(
        _sce_body,
        out_type=(jax.ShapeDtypeStruct((N_SUB, 64), jnp.float32),
                  jax.ShapeDtypeStruct((N_SUB, 64), jnp.float32)),
        mesh=_mesh(),
        compiler_params=_SC_PARAMS,
        scratch_types=[
            pltpu.VMEM((128,), jnp.int32),
            pltpu.VMEM((128,), jnp.int32),
            pltpu.VMEM((64,), jnp.int32),
            pltpu.VMEM((64,), jnp.int32),
            pltpu.VMEM((128, 128), jnp.float32),
            pltpu.VMEM((128, 64), jnp.float32),
            pltpu.VMEM((128, 64), jnp.float32),
            pltpu.VMEM((128, 64), jnp.float32),
            pltpu.SemaphoreType.DMA,
            pltpu.SemaphoreType.DMA,
            pltpu.SemaphoreType.DMA,
            pltpu.SemaphoreType.DMA,
            pltpu.SemaphoreType.DMA,
            pltpu.VMEM_SHARED((N_SUB, 64), jnp.float32),
        ],
    )
    return f(kv0, kv1, qs0, qs1, ef0, ef1, src_e, dst_e, zeros)


def _fin_body(a0_ref, a1_ref, zs_ref, z_ref, zp_ref):
    a0 = a0_ref[...]
    a1 = a1_ref[...]
    s0 = a0[:, HEAD_DIM:HEAD_DIM + 1]
    s1 = a1[:, HEAD_DIM:HEAD_DIM + 1]
    h0 = a0[:, :HEAD_DIM] / jnp.where(s0 > 0, s0, 1.0)
    h1 = a1[:, :HEAD_DIM] / jnp.where(s1 > 0, s1, 1.0)
    z = jnp.concatenate([h0, h1], axis=1) + zs_ref[...]
    z_ref[...] = z
    zp_ref[...] = jnp.concatenate(
        [z, jnp.zeros((z.shape[0], 12), jnp.float32)], axis=1)


def _finalize(acc0, acc1, zs):
    nblk = N_SUB // NODE_BLK
    return pl.pallas_call(
        _fin_body,
        grid=(nblk,),
        in_specs=[
            pl.BlockSpec((NODE_BLK, 64), lambda i: (i, 0)),
            pl.BlockSpec((NODE_BLK, 64), lambda i: (i, 0)),
            pl.BlockSpec((NODE_BLK, EMB_DIM), lambda i: (i, 0)),
        ],
        out_specs=(pl.BlockSpec((NODE_BLK, EMB_DIM), lambda i: (i, 0)),
                   pl.BlockSpec((NODE_BLK, 112), lambda i: (i, 0))),
        out_shape=(jax.ShapeDtypeStruct((N_SUB, EMB_DIM), jnp.float32),
                   jax.ShapeDtypeStruct((N_SUB, 112), jnp.float32)),
    )(acc0, acc1, zs)


def _sc2_body(z_hbm, loc_hbm, zsel_hbm, idx_v, row_v, sem):
    c = lax.axis_index("c")
    s = lax.axis_index("s")
    wid = s * 2 + c

    def chunk(g, _):
        base = (wid * 3 + g) * 128
        pltpu.sync_copy(loc_hbm.at[pl.ds(base, 128)], idx_v)
        pltpu.async_copy(z_hbm.at[idx_v], row_v, sem).wait()
        pltpu.sync_copy(row_v, zsel_hbm.at[pl.ds(base, 128), :])
        return _
    lax.fori_loop(0, 3, chunk, None, unroll=False)


def _sc2(z, loc):
    f = pl.kernel(
        _sc2_body,
        out_type=jax.ShapeDtypeStruct((NQ, 112), jnp.float32),
        mesh=_mesh(),
        compiler_params=_SC_PARAMS,
        scratch_types=[
            pltpu.VMEM((128,), jnp.int32),
            pltpu.VMEM((128, 112), jnp.float32),
            pltpu.SemaphoreType.DMA,
        ],
    )
    return f(z, loc)


def _linkpred_body(zs_ref, zd_ref, zn_ref, Wls_ref, bls_ref, Wld_ref, bld_ref,
                   Wlf_ref, blf_ref, pos_ref, neg_ref):
    zsv = zs_ref[...][:, :EMB_DIM]
    zdv = zd_ref[...][:, :EMB_DIM]
    znv = zn_ref[...][:, :EMB_DIM]
    a = jnp.dot(zsv, Wls_ref[...], preferred_element_type=jnp.float32) + bls_ref[...]
    hd = jnp.dot(zdv, Wld_ref[...], preferred_element_type=jnp.float32) + bld_ref[...]
    hn = jnp.dot(znv, Wld_ref[...], preferred_element_type=jnp.float32) + bld_ref[...]
    h = jnp.maximum(a + hd, 0.0)
    hneg = jnp.maximum(a + hn, 0.0)
    pos_ref[...] = jnp.dot(h, Wlf_ref[...], preferred_element_type=jnp.float32) + blf_ref[...]
    neg_ref[...] = jnp.dot(hneg, Wlf_ref[...], preferred_element_type=jnp.float32) + blf_ref[...]


def _linkpred(zsel, Wls, bls, Wld, bld, Wlf, blf):
    out_shape = (jax.ShapeDtypeStruct((B, 1), jnp.float32),
                 jax.ShapeDtypeStruct((B, 1), jnp.float32))
    return pl.pallas_call(
        _linkpred_body,
        out_shape=out_shape,
    )(zsel[:B], zsel[B:2 * B], zsel[2 * B:], Wls, bls[None, :], Wld,
      bld[None, :], Wlf, blf[None, :])


def kernel(memory, last_update, t, msg, w_time, b_time, Wq, bq, Wk, bk, Wv, bv,
           We, be, Ws, bs, Wls, bls, Wld, bld, Wlf, blf,
           n_id, edge_index, src, dst, neg_dst):
    src_e = edge_index[0]
    dst_e = edge_index[1]
    queries = jnp.concatenate([src, dst, neg_dst])

    memp = _pad112(memory)
    if _DBG_JAX_SC1A:
        z0 = jnp.pad(memory[n_id], ((0, 0), (0, 12)))
        assoc = jnp.zeros((NUM_NODES,), jnp.int32).at[n_id].set(
            jnp.arange(N_SUB, dtype=jnp.int32))
        loc = assoc[queries]
    else:
        z0, loc = _sc1a(memp, n_id, queries)
        if _DBG_Z0_JAX:
            z0 = jnp.pad(memory[n_id], ((0, 0), (0, 12)))
    if _DBG_JAX_SC1B:
        rel_t = last_update[n_id][src_e] - t
    else:
        rel_t = _sc1b(last_update, n_id, src_e, t)
    kv0, kv1, qs0, qs1, zs = _nodeproj(z0, Wq, bq, Wk, bk, Wv, bv, Ws, bs)
    ef0, ef1 = _edgefeat(rel_t, msg, w_time, b_time, We, be)
    if _DBG_JAX_SCE:
        def sce_ref(kv, qs, ef):
            kk = kv[:, :64][src_e]
            vv = kv[:, 64:][src_e]
            qq = qs[dst_e]
            p = jnp.exp((qq * (kk + ef)).sum(-1))
            return jax.ops.segment_sum((vv + ef) * p[:, None], dst_e,
                                       num_segments=N_SUB)
        acc0 = sce_ref(kv0, qs0, ef0)
        acc1 = sce_ref(kv1, qs1, ef1)
    else:
        acc0, acc1 = _sce(kv0, kv1, qs0, qs1, ef0, ef1, src_e, dst_e)
    z, zp = _finalize(acc0, acc1, zs)
    if _DBG_JAX_SC2:
        zsel = zp[loc]
    else:
        zsel = _sc2(zp, loc)
    pos_out, neg_out = _linkpred(zsel, Wls, bls, Wld, bld, Wlf, blf)
    return (z, pos_out, neg_out)
